# Initial kernel scaffold; baseline (speedup 1.0000x reference)
import jax
import jax.numpy as jnp
from jax.experimental import pallas as pl


def kernel(x, weight):
    raise NotImplementedError("write your pallas kernel here")


# speedup vs baseline: 2.8053x; 2.8053x over previous
import jax
import jax.numpy as jnp
from jax import lax
from jax.experimental import pallas as pl
from jax.experimental.pallas import tpu as pltpu
from jax.experimental.pallas import tpu_sc as plsc

B = 16384
K = 50
D = 32
NC, NS, L = 2, 16, 16
NW = NC * NS
SPW = B // NW
CS = 4
IDXC = CS * K
NCH = SPW // CS

_mesh = plsc.VectorSubcoreMesh(core_axis_name="c", subcore_axis_name="s")


def _body(x_hbm, w_hbm, out_hbm, idx_all, rows0, rows1, out_buf, sem0, sem1):
    wid = lax.axis_index("s") * NC + lax.axis_index("c")
    base = wid * (SPW * K)

    pltpu.sync_copy(x_hbm.at[pl.ds(base, SPW * K)], idx_all)

    def start(c, buf, sem):
        off = pl.multiple_of(c * IDXC, 8)
        pltpu.async_copy(w_hbm.at[idx_all.at[pl.ds(off, IDXC)]], buf, sem)

    def wait(buf, sem):
        pltpu.make_async_copy(w_hbm.at[idx_all.at[pl.ds(0, IDXC)]], buf, sem).wait()

    def reduce_chunk(buf, c):
        for s in range(CS):
            a0 = buf[s * K, 0:L]
            a1 = buf[s * K, L:D]
            for j in range(1, K):
                a0 = a0 + buf[s * K + j, 0:L]
                a1 = a1 + buf[s * K + j, L:D]
            row = c * CS + s
            out_buf[row, 0:L] = a0
            out_buf[row, L:D] = a1

    start(0, rows0, sem0)

    def pair(i, carry):
        c0 = i * 2
        start(c0 + 1, rows1, sem1)
        wait(rows0, sem0)
        reduce_chunk(rows0, c0)
        start(c0 + 2, rows0, sem0)
        wait(rows1, sem1)
        reduce_chunk(rows1, c0 + 1)
        return carry

    lax.fori_loop(0, NCH // 2 - 1, pair, 0)
    start(NCH - 1, rows1, sem1)
    wait(rows0, sem0)
    reduce_chunk(rows0, NCH - 2)
    wait(rows1, sem1)
    reduce_chunk(rows1, NCH - 1)

    pltpu.sync_copy(out_buf, out_hbm.at[pl.ds(wid * SPW, SPW)])


_emb_sum = pl.kernel(
    _body,
    out_type=jax.ShapeDtypeStruct((B, D), jnp.float32),
    mesh=_mesh,
    scratch_types=[
        pltpu.VMEM((SPW * K,), jnp.int32),
        pltpu.VMEM((IDXC, D), jnp.float32),
        pltpu.VMEM((IDXC, D), jnp.float32),
        pltpu.VMEM((SPW, D), jnp.float32),
        pltpu.SemaphoreType.DMA,
        pltpu.SemaphoreType.DMA,
    ],
    compiler_params=pltpu.CompilerParams(use_tc_tiling_on_sc=False),
)


def kernel(x, weight):
    return _emb_sum(x.reshape(-1), weight)
